# baseline (speedup 1.0000x reference)
import functools

import jax
import jax.numpy as jnp
from jax import lax
from jax.experimental import pallas as pl
from jax.experimental.pallas import tpu as pltpu
from jax.experimental.pallas import tpu_sc as plsc

_BATCH = 16384

_info = plsc.get_sparse_core_info()
_NC = _info.num_cores
_NS = _info.num_subcores
_NW = _NC * _NS
_BPW = _BATCH // _NW


def _bias_body(u_hbm, i_hbm, ub_hbm, ib_hbm, ub_out, ib_out,
               uidx_v, iidx_v, u2_v, i2_v, usem, isem):
    wid = lax.axis_index("s") * _NC + lax.axis_index("c")
    base = wid * _BPW

    pltpu.sync_copy(u_hbm.at[pl.ds(base, _BPW)], uidx_v)
    pltpu.sync_copy(i_hbm.at[pl.ds(base, _BPW)], iidx_v)

    ucopy = pltpu.async_copy(ub_hbm.at[uidx_v], u2_v, usem)
    icopy = pltpu.async_copy(ib_hbm.at[iidx_v], i2_v, isem)
    ucopy.wait()
    icopy.wait()

    pltpu.sync_copy(u2_v, ub_out.at[pl.ds(base, _BPW)])
    pltpu.sync_copy(i2_v, ib_out.at[pl.ds(base, _BPW)])


@jax.jit
def _bias_sc(u_id, i_id, ub, ib):
    return pl.kernel(
        _bias_body,
        out_type=(
            jax.ShapeDtypeStruct((_BATCH, 1), jnp.float32),
            jax.ShapeDtypeStruct((_BATCH, 1), jnp.float32),
        ),
        mesh=plsc.VectorSubcoreMesh(core_axis_name="c", subcore_axis_name="s"),
        compiler_params=pltpu.CompilerParams(use_tc_tiling_on_sc=False),
        scratch_types=[
            pltpu.VMEM((_BPW,), jnp.int32),
            pltpu.VMEM((_BPW,), jnp.int32),
            pltpu.VMEM((_BPW, 1), jnp.float32),
            pltpu.VMEM((_BPW, 1), jnp.float32),
            pltpu.SemaphoreType.DMA,
            pltpu.SemaphoreType.DMA,
        ],
    )(u_id, i_id, ub, ib)


def kernel(u_id, i_id, user_bias, item_bias, global_bias):
    u_b, i_b = _bias_sc(
        u_id.astype(jnp.int32),
        i_id.astype(jnp.int32),
        user_bias,
        item_bias,
    )
    return (u_b + i_b).reshape(-1) + global_bias


# speedup vs baseline: 16.3030x; 16.3030x over previous
import functools

import jax
import jax.numpy as jnp
from jax import lax
from jax.experimental import pallas as pl
from jax.experimental.pallas import tpu as pltpu
from jax.experimental.pallas import tpu_sc as plsc

_BATCH = 16384

_info = plsc.get_sparse_core_info()
_NC = _info.num_cores
_NS = _info.num_subcores
_NW = _NC * _NS
_BPW = _BATCH // _NW


def _gather_body(idx_hbm, tab_hbm, out_hbm, idx_v, rows_v, sem):
    wid = lax.axis_index("s") * _NC + lax.axis_index("c")
    base = wid * _BPW
    pltpu.sync_copy(idx_hbm.at[pl.ds(base, _BPW)], idx_v)
    pltpu.async_copy(tab_hbm.at[idx_v], rows_v, sem).wait()
    pltpu.sync_copy(rows_v, out_hbm.at[pl.ds(base, _BPW)])


_gather_sc = pl.kernel(
    _gather_body,
    out_type=jax.ShapeDtypeStruct((_BATCH,), jnp.float32),
    mesh=plsc.VectorSubcoreMesh(core_axis_name="c", subcore_axis_name="s"),
    scratch_types=[
        pltpu.VMEM((_BPW,), jnp.int32),
        pltpu.VMEM((_BPW,), jnp.float32),
        pltpu.SemaphoreType.DMA,
    ],
)


@jax.jit
def _bias(u_id, i_id, user_bias, item_bias, global_bias):
    u_b = _gather_sc(u_id, user_bias.reshape(-1))
    i_b = _gather_sc(i_id, item_bias.reshape(-1))
    return u_b + i_b + global_bias


def kernel(u_id, i_id, user_bias, item_bias, global_bias):
    return _bias(u_id.astype(jnp.int32), i_id.astype(jnp.int32),
                 user_bias, item_bias, global_bias)


# speedup vs baseline: 48.1690x; 2.9546x over previous
import functools

import jax
import jax.numpy as jnp
from jax import lax
from jax.experimental import pallas as pl
from jax.experimental.pallas import tpu as pltpu
from jax.experimental.pallas import tpu_sc as plsc

_BATCH = 16384
_LANES = 16
_PAD_TO = 1000448

_info = plsc.get_sparse_core_info()
_NC = _info.num_cores
_NS = _info.num_subcores
_NW = _NC * _NS
_BPW = _BATCH // _NW


def _bias_body(u_hbm, i_hbm, ub_hbm, ib_hbm, gb_hbm, out_hbm,
               uidx_v, iidx_v, urows_v, irows_v, gb_v, usem, isem):
    wid = lax.axis_index("s") * _NC + lax.axis_index("c")
    base = wid * _BPW

    pltpu.sync_copy(u_hbm.at[pl.ds(base, _BPW)], uidx_v)
    pltpu.sync_copy(i_hbm.at[pl.ds(base, _BPW)], iidx_v)

    ucopy = pltpu.async_copy(ub_hbm.at[uidx_v], urows_v, usem)
    icopy = pltpu.async_copy(ib_hbm.at[iidx_v], irows_v, isem)

    pltpu.sync_copy(gb_hbm, gb_v)
    g = gb_v[...]

    ucopy.wait()
    icopy.wait()

    for j in range(_BPW // _LANES):
        sl = pl.ds(j * _LANES, _LANES)
        urows_v[sl] = urows_v[sl] + irows_v[sl] + g

    pltpu.sync_copy(urows_v, out_hbm.at[pl.ds(base, _BPW)])


@jax.jit
def _bias(u_id, i_id, user_bias, item_bias, global_bias):
    ub = jnp.pad(user_bias, ((0, _PAD_TO - user_bias.shape[0]), (0, 0))).reshape(-1)
    ib = jnp.pad(item_bias, ((0, _PAD_TO - item_bias.shape[0]), (0, 0))).reshape(-1)
    gb = jnp.broadcast_to(global_bias, (_LANES,))
    return pl.kernel(
        _bias_body,
        out_type=jax.ShapeDtypeStruct((_BATCH,), jnp.float32),
        mesh=plsc.VectorSubcoreMesh(core_axis_name="c", subcore_axis_name="s"),
        scratch_types=[
            pltpu.VMEM((_BPW,), jnp.int32),
            pltpu.VMEM((_BPW,), jnp.int32),
            pltpu.VMEM((_BPW,), jnp.float32),
            pltpu.VMEM((_BPW,), jnp.float32),
            pltpu.VMEM((_LANES,), jnp.float32),
            pltpu.SemaphoreType.DMA,
            pltpu.SemaphoreType.DMA,
        ],
    )(u_id, i_id, ub, ib, gb)


def kernel(u_id, i_id, user_bias, item_bias, global_bias):
    return _bias(u_id.astype(jnp.int32), i_id.astype(jnp.int32),
                 user_bias, item_bias, global_bias)


# speedup vs baseline: 50.0732x; 1.0395x over previous
import functools

import jax
import jax.numpy as jnp
from jax import lax
from jax.experimental import pallas as pl
from jax.experimental.pallas import tpu as pltpu
from jax.experimental.pallas import tpu_sc as plsc

_BATCH = 16384
_LANES = 16
_PAD_TO = 1000448
_GB_AT = 1000000

_info = plsc.get_sparse_core_info()
_NC = _info.num_cores
_NS = _info.num_subcores
_NW = _NC * _NS
_BPW = _BATCH // _NW


def _bias_body(u_hbm, i_hbm, ub_hbm, ib_hbm, out_hbm,
               uidx_v, iidx_v, urows_v, irows_v, gb_v, usem, isem):
    wid = lax.axis_index("s") * _NC + lax.axis_index("c")
    base = wid * _BPW

    pltpu.sync_copy(u_hbm.at[pl.ds(base, _BPW)], uidx_v)
    pltpu.sync_copy(i_hbm.at[pl.ds(base, _BPW)], iidx_v)

    ucopy = pltpu.async_copy(ub_hbm.at[uidx_v], urows_v, usem)
    icopy = pltpu.async_copy(ib_hbm.at[iidx_v], irows_v, isem)

    pltpu.sync_copy(ub_hbm.at[pl.ds(_GB_AT, _LANES)], gb_v)
    g = gb_v[...]

    ucopy.wait()
    icopy.wait()

    for j in range(_BPW // _LANES):
        sl = pl.ds(j * _LANES, _LANES)
        urows_v[sl] = urows_v[sl] + irows_v[sl] + g

    pltpu.sync_copy(urows_v, out_hbm.at[pl.ds(base, _BPW)])


@jax.jit
def _bias(u_id, i_id, user_bias, item_bias, global_bias):
    ub = jnp.pad(user_bias, ((0, _PAD_TO - user_bias.shape[0]), (0, 0)),
                 constant_values=global_bias[0]).reshape(-1)
    ib = jnp.pad(item_bias, ((0, _PAD_TO - item_bias.shape[0]), (0, 0))).reshape(-1)
    return pl.kernel(
        _bias_body,
        out_type=jax.ShapeDtypeStruct((_BATCH,), jnp.float32),
        mesh=plsc.VectorSubcoreMesh(core_axis_name="c", subcore_axis_name="s"),
        scratch_types=[
            pltpu.VMEM((_BPW,), jnp.int32),
            pltpu.VMEM((_BPW,), jnp.int32),
            pltpu.VMEM((_BPW,), jnp.float32),
            pltpu.VMEM((_BPW,), jnp.float32),
            pltpu.VMEM((_LANES,), jnp.float32),
            pltpu.SemaphoreType.DMA,
            pltpu.SemaphoreType.DMA,
        ],
    )(u_id, i_id, ub, ib)


def kernel(u_id, i_id, user_bias, item_bias, global_bias):
    return _bias(u_id.astype(jnp.int32), i_id.astype(jnp.int32),
                 user_bias, item_bias, global_bias)


# speedup vs baseline: 50.6122x; 1.0108x over previous
import jax
import jax.numpy as jnp
from jax import lax
from jax.experimental import pallas as pl
from jax.experimental.pallas import tpu as pltpu
from jax.experimental.pallas import tpu_sc as plsc

_BATCH = 16384
_LANES = 16
_PAD_TO = 1000448
_GB_AT = 1000000

_info = plsc.get_sparse_core_info()
_NC = _info.num_cores
_NS = _info.num_subcores
_NW = _NC * _NS
_BPW = _BATCH // _NW


def _bias_body(u_hbm, i_hbm, ub_hbm, ib_hbm, out_hbm,
               uidx_v, iidx_v, urows_v, irows_v, gb_v, usem, isem):
    wid = lax.axis_index("s") * _NC + lax.axis_index("c")
    base = wid * _BPW

    uicopy = pltpu.async_copy(u_hbm.at[pl.ds(base, _BPW)], uidx_v, usem)
    iicopy = pltpu.async_copy(i_hbm.at[pl.ds(base, _BPW)], iidx_v, isem)

    pltpu.sync_copy(ub_hbm.at[pl.ds(_GB_AT, _LANES)], gb_v)
    g = gb_v[...]

    uicopy.wait()
    ucopy = pltpu.async_copy(ub_hbm.at[uidx_v], urows_v, usem)
    iicopy.wait()
    icopy = pltpu.async_copy(ib_hbm.at[iidx_v], irows_v, isem)

    ucopy.wait()
    icopy.wait()

    for j in range(_BPW // _LANES):
        sl = pl.ds(j * _LANES, _LANES)
        urows_v[sl] = urows_v[sl] + irows_v[sl] + g

    pltpu.sync_copy(urows_v, out_hbm.at[pl.ds(base, _BPW)])


@jax.jit
def _bias(u_id, i_id, user_bias, item_bias, global_bias):
    ub = jnp.pad(user_bias, ((0, _PAD_TO - user_bias.shape[0]), (0, 0)),
                 constant_values=global_bias[0]).reshape(-1)
    ib = jnp.pad(item_bias, ((0, _PAD_TO - item_bias.shape[0]), (0, 0))).reshape(-1)
    return pl.kernel(
        _bias_body,
        out_type=jax.ShapeDtypeStruct((_BATCH,), jnp.float32),
        mesh=plsc.VectorSubcoreMesh(core_axis_name="c", subcore_axis_name="s"),
        scratch_types=[
            pltpu.VMEM((_BPW,), jnp.int32),
            pltpu.VMEM((_BPW,), jnp.int32),
            pltpu.VMEM((_BPW,), jnp.float32),
            pltpu.VMEM((_BPW,), jnp.float32),
            pltpu.VMEM((_LANES,), jnp.float32),
            pltpu.SemaphoreType.DMA,
            pltpu.SemaphoreType.DMA,
        ],
    )(u_id, i_id, ub, ib)


def kernel(u_id, i_id, user_bias, item_bias, global_bias):
    return _bias(u_id.astype(jnp.int32), i_id.astype(jnp.int32),
                 user_bias, item_bias, global_bias)
